# baseline (speedup 1.0000x reference)
import functools

import jax
import jax.numpy as jnp
from jax import lax
from jax.experimental import pallas as pl
from jax.experimental.pallas import tpu as pltpu
from jax.experimental.pallas import tpu_sc as plsc

SIGMA2X2 = 2.0 * 20.0 * 20.0
KB = 4096
GRP = 32
TOPK = 64


def _score_body(nk_real, q_ref, k_ref, w_ref, gmax_ref):
    kb = pl.program_id(1)
    q = q_ref[...]
    QB = q.shape[0]
    q2 = jnp.sum(q * q, axis=1, keepdims=True)
    for r in range(KB // 128):
        k = k_ref[r * 128:(r + 1) * 128, :]
        qk = jax.lax.dot_general(q, k, (((1,), (1,)), ((), ())),
                                 preferred_element_type=jnp.float32)
        k2 = jnp.sum(k * k, axis=1)
        d2 = q2 + k2[None, :] - 2.0 * qk
        d2 = jnp.maximum(d2, 0.0)
        w = jnp.exp(-d2 / SIGMA2X2)
        col = kb * KB + r * 128 + jax.lax.broadcasted_iota(
            jnp.int32, w.shape, 1)
        w = jnp.where(col < nk_real, w, 0.0)
        w_ref[:, r, :] = w
        gmax_ref[:, r * 4:(r + 1) * 4] = jnp.max(
            w.reshape(QB, 4, GRP), axis=2)


def _scores(queries, keys_p, nk_real):
    Q, D = queries.shape
    KP = keys_p.shape[0]
    nkb = KP // KB
    QB = min(256, Q)
    return pl.pallas_call(
        functools.partial(_score_body, nk_real),
        grid=(Q // QB, nkb),
        in_specs=[
            pl.BlockSpec((QB, D), lambda qb, kb: (qb, 0)),
            pl.BlockSpec((KB, D), lambda qb, kb: (kb, 0)),
        ],
        out_specs=[
            pl.BlockSpec((QB, KB // 128, 128), lambda qb, kb: (qb, kb, 0)),
            pl.BlockSpec((QB, KB // GRP), lambda qb, kb: (qb, kb)),
        ],
        out_shape=[
            jax.ShapeDtypeStruct((Q, KP // 128, 128), jnp.float32),
            jax.ShapeDtypeStruct((Q, KP // GRP), jnp.float32),
        ],
    )(queries, keys_p)


def _topgroups_body(G, gmax_ref, grow_ref, sc_ref):
    sc_ref[...] = gmax_ref[...]
    Q = gmax_ref.shape[0]
    qoff = jax.lax.broadcasted_iota(jnp.int32, (Q, 1), 0) * G
    slot = jax.lax.broadcasted_iota(jnp.int32, (Q, TOPK), 1)

    def body(t, acc):
        a = sc_ref[...]
        m = jnp.max(a, axis=1, keepdims=True)
        gi = jax.lax.broadcasted_iota(jnp.int32, a.shape, 1)
        idx = jnp.min(jnp.where(a == m, gi, jnp.int32(2**30)), axis=1,
                      keepdims=True)
        sc_ref[...] = jnp.where(gi == idx, -1.0, a)
        return jnp.where(slot == t, idx + qoff, acc)

    grow_ref[...] = jax.lax.fori_loop(
        0, TOPK, body, jnp.zeros((Q, TOPK), jnp.int32))


def _topgroups(gmax):
    Q, G = gmax.shape
    return pl.pallas_call(
        functools.partial(_topgroups_body, G),
        out_shape=jax.ShapeDtypeStruct((Q, TOPK), jnp.int32),
        scratch_shapes=[pltpu.VMEM((Q, G), jnp.float32)],
    )(gmax)


def _select_body(G, QB, cand4_ref, grow_ref, topi_ref, wn_ref, sc_ref):
    qb = pl.program_id(0)
    grow = grow_ref[...]
    q = qb * QB + jax.lax.broadcasted_iota(jnp.int32, (QB, TOPK), 0)
    g = grow - q * G
    gmod = jnp.bitwise_and(grow, 3)
    cand4 = cand4_ref[...]
    acc = jnp.full((QB, TOPK, GRP), -1.0, jnp.float32)
    for c in range(4):
        acc = jnp.where(gmod[:, :, None] == c,
                        cand4[:, :, c * GRP:(c + 1) * GRP], acc)
    C = TOPK * GRP
    sc_ref[...] = acc.reshape(QB, C)
    kidx3 = g[:, :, None] * GRP + jax.lax.broadcasted_iota(
        jnp.int32, (QB, TOPK, GRP), 2)
    kidx = kidx3.reshape(QB, C)
    slot = jax.lax.broadcasted_iota(jnp.int32, (QB, TOPK), 1)

    def body(t, carry):
        acc_i, acc_w = carry
        a = sc_ref[...]
        m = jnp.max(a, axis=1, keepdims=True)
        ki = jnp.min(jnp.where(a == m, kidx, jnp.int32(2**30)), axis=1,
                     keepdims=True)
        sc_ref[...] = jnp.where(kidx == ki, -1.0, a)
        return (jnp.where(slot == t, ki, acc_i),
                jnp.where(slot == t, m, acc_w))

    ti, tw = jax.lax.fori_loop(
        0, TOPK, body,
        (jnp.zeros((QB, TOPK), jnp.int32), jnp.zeros((QB, TOPK), jnp.float32)))
    topi_ref[...] = ti
    wn_ref[...] = tw / jnp.maximum(jnp.sum(tw, axis=1, keepdims=True), 1e-12)


def _select(cand4, grow, G):
    Q = cand4.shape[0]
    QB = min(256, Q)
    return pl.pallas_call(
        functools.partial(_select_body, G, QB),
        grid=(Q // QB,),
        in_specs=[
            pl.BlockSpec((QB, TOPK, 128), lambda qb: (qb, 0, 0)),
            pl.BlockSpec((QB, TOPK), lambda qb: (qb, 0)),
        ],
        out_specs=[
            pl.BlockSpec((QB, TOPK), lambda qb: (qb, 0)),
            pl.BlockSpec((QB, TOPK), lambda qb: (qb, 0)),
        ],
        out_shape=[
            jax.ShapeDtypeStruct((Q, TOPK), jnp.int32),
            jax.ShapeDtypeStruct((Q, TOPK), jnp.float32),
        ],
        scratch_shapes=[pltpu.VMEM((QB, TOPK * GRP), jnp.float32)],
    )(cand4, grow)


def _sc_gather(table, idx):
    NC, NS = 2, 16
    NW = NC * NS
    BR, D = idx.shape[0] * 128, table.shape[1]
    rpw = BR // NW
    nch = rpw // 128
    mesh = plsc.VectorSubcoreMesh(core_axis_name="c", subcore_axis_name="s")

    @functools.partial(
        pl.kernel, mesh=mesh,
        out_type=jax.ShapeDtypeStruct((BR, D), jnp.float32),
        scratch_types=[
            pltpu.VMEM((nch, 128), jnp.int32),
            pltpu.VMEM((128, D), jnp.float32),
            pltpu.SemaphoreType.DMA,
        ],
    )
    def k(table_hbm, idx_hbm, out_hbm, idx_v, rows_v, sem):
        wid = lax.axis_index("s") * NC + lax.axis_index("c")
        pltpu.sync_copy(idx_hbm.at[pl.ds(wid * nch, nch)], idx_v)
        for j in range(nch):
            pltpu.async_copy(table_hbm.at[idx_v.at[j]], rows_v, sem).wait()
            pltpu.sync_copy(
                rows_v, out_hbm.at[pl.ds(wid * rpw + j * 128, 128)])

    return k(table, idx)


def kernel(queries, keys, values):
    Q, D = queries.shape
    NK = keys.shape[0]
    KP = ((NK + KB - 1) // KB) * KB
    keys_p = jnp.pad(keys, ((0, KP - NK), (0, 0)))
    w, gmax = _scores(queries, keys_p, NK)
    G = KP // GRP
    grow = _topgroups(gmax)
    cand4 = w.reshape(Q * (KP // 128), 128)[
        (grow >> 2).reshape(-1)].reshape(Q, TOPK, 128)
    topi, wn = _select(cand4, grow, G)
    gathered = _sc_gather(values, topi.reshape(Q * TOPK // 128, 128))
    out = jnp.einsum("qt,qtd->qd", wn, gathered.reshape(Q, TOPK, D))
    return out, topi


# speedup vs baseline: 1.6867x; 1.6867x over previous
import functools

import jax
import jax.numpy as jnp
from jax import lax
from jax.experimental import pallas as pl
from jax.experimental.pallas import tpu as pltpu
from jax.experimental.pallas import tpu_sc as plsc

SIGMA2X2 = 2.0 * 20.0 * 20.0
KB = 4096
GRP = 32
TOPK = 64


def _score_body(nk_real, q_ref, k_ref, w_ref, gmax_ref):
    kb = pl.program_id(1)
    q = q_ref[...]
    k = k_ref[...]
    qk = jax.lax.dot_general(q, k, (((1,), (1,)), ((), ())),
                             preferred_element_type=jnp.float32)
    q2 = jnp.sum(q * q, axis=1, keepdims=True)
    k2 = jnp.sum(k * k, axis=1)
    d2 = q2 + k2[None, :] - 2.0 * qk
    d2 = jnp.maximum(d2, 0.0)
    w = jnp.exp(-d2 / SIGMA2X2)
    col = kb * KB + jax.lax.broadcasted_iota(jnp.int32, w.shape, 1)
    w = jnp.where(col < nk_real, w, 0.0)
    w_ref[...] = w.reshape(w.shape[0], KB // 128, 128)
    gmax_ref[...] = jnp.max(w.reshape(w.shape[0], KB // GRP, GRP), axis=2)


def _scores(queries, keys_p, nk_real):
    Q, D = queries.shape
    KP = keys_p.shape[0]
    nkb = KP // KB
    QB = min(256, Q)
    return pl.pallas_call(
        functools.partial(_score_body, nk_real),
        grid=(Q // QB, nkb),
        in_specs=[
            pl.BlockSpec((QB, D), lambda qb, kb: (qb, 0)),
            pl.BlockSpec((KB, D), lambda qb, kb: (kb, 0)),
        ],
        out_specs=[
            pl.BlockSpec((QB, KB // 128, 128), lambda qb, kb: (qb, kb, 0)),
            pl.BlockSpec((QB, KB // GRP), lambda qb, kb: (qb, kb)),
        ],
        out_shape=[
            jax.ShapeDtypeStruct((Q, KP // 128, 128), jnp.float32),
            jax.ShapeDtypeStruct((Q, KP // GRP), jnp.float32),
        ],
    )(queries, keys_p)


def _topgroups_body(G, gmax_ref, grow_ref, sc_ref):
    sc_ref[...] = gmax_ref[...]
    Q = gmax_ref.shape[0]
    qoff = jax.lax.broadcasted_iota(jnp.int32, (Q, 1), 0) * G
    slot = jax.lax.broadcasted_iota(jnp.int32, (Q, TOPK), 1)

    def body(t, acc):
        a = sc_ref[...]
        m = jnp.max(a, axis=1, keepdims=True)
        gi = jax.lax.broadcasted_iota(jnp.int32, a.shape, 1)
        idx = jnp.min(jnp.where(a == m, gi, jnp.int32(2**30)), axis=1,
                      keepdims=True)
        sc_ref[...] = jnp.where(gi == idx, -1.0, a)
        return jnp.where(slot == t, idx + qoff, acc)

    grow_ref[...] = jax.lax.fori_loop(
        0, TOPK, body, jnp.zeros((Q, TOPK), jnp.int32))


def _topgroups(gmax):
    Q, G = gmax.shape
    return pl.pallas_call(
        functools.partial(_topgroups_body, G),
        out_shape=jax.ShapeDtypeStruct((Q, TOPK), jnp.int32),
        scratch_shapes=[pltpu.VMEM((Q, G), jnp.float32)],
    )(gmax)


def _select_body(G, QB, cand4_ref, grow_ref, topi_ref, wn_ref, sc_ref):
    qb = pl.program_id(0)
    grow = grow_ref[...]
    q = qb * QB + jax.lax.broadcasted_iota(jnp.int32, (QB, TOPK), 0)
    g = grow - q * G
    gmod = jnp.bitwise_and(grow, 3)
    cand4 = cand4_ref[...]
    acc = jnp.full((QB, TOPK, GRP), -1.0, jnp.float32)
    for c in range(4):
        acc = jnp.where(gmod[:, :, None] == c,
                        cand4[:, :, c * GRP:(c + 1) * GRP], acc)
    C = TOPK * GRP
    sc_ref[...] = acc.reshape(QB, C)
    kidx3 = g[:, :, None] * GRP + jax.lax.broadcasted_iota(
        jnp.int32, (QB, TOPK, GRP), 2)
    kidx = kidx3.reshape(QB, C)
    slot = jax.lax.broadcasted_iota(jnp.int32, (QB, TOPK), 1)

    def body(t, carry):
        acc_i, acc_w = carry
        a = sc_ref[...]
        m = jnp.max(a, axis=1, keepdims=True)
        ki = jnp.min(jnp.where(a == m, kidx, jnp.int32(2**30)), axis=1,
                     keepdims=True)
        sc_ref[...] = jnp.where(kidx == ki, -1.0, a)
        return (jnp.where(slot == t, ki, acc_i),
                jnp.where(slot == t, m, acc_w))

    ti, tw = jax.lax.fori_loop(
        0, TOPK, body,
        (jnp.zeros((QB, TOPK), jnp.int32), jnp.zeros((QB, TOPK), jnp.float32)))
    topi_ref[...] = ti
    wn_ref[...] = tw / jnp.maximum(jnp.sum(tw, axis=1, keepdims=True), 1e-12)


def _select(cand4, grow, G):
    Q = cand4.shape[0]
    QB = min(256, Q)
    return pl.pallas_call(
        functools.partial(_select_body, G, QB),
        grid=(Q // QB,),
        in_specs=[
            pl.BlockSpec((QB, TOPK, 128), lambda qb: (qb, 0, 0)),
            pl.BlockSpec((QB, TOPK), lambda qb: (qb, 0)),
        ],
        out_specs=[
            pl.BlockSpec((QB, TOPK), lambda qb: (qb, 0)),
            pl.BlockSpec((QB, TOPK), lambda qb: (qb, 0)),
        ],
        out_shape=[
            jax.ShapeDtypeStruct((Q, TOPK), jnp.int32),
            jax.ShapeDtypeStruct((Q, TOPK), jnp.float32),
        ],
        scratch_shapes=[pltpu.VMEM((QB, TOPK * GRP), jnp.float32)],
    )(cand4, grow)


def _sc_gather(table, idx):
    NC, NS = 2, 16
    NW = NC * NS
    BR, D = idx.shape[0] * 128, table.shape[1]
    rpw = BR // NW
    nch = rpw // 128
    mesh = plsc.VectorSubcoreMesh(core_axis_name="c", subcore_axis_name="s")

    @functools.partial(
        pl.kernel, mesh=mesh,
        out_type=jax.ShapeDtypeStruct((BR, D), jnp.float32),
        scratch_types=[
            pltpu.VMEM((nch, 128), jnp.int32),
            pltpu.VMEM((128, D), jnp.float32),
            pltpu.SemaphoreType.DMA,
        ],
    )
    def k(table_hbm, idx_hbm, out_hbm, idx_v, rows_v, sem):
        wid = lax.axis_index("s") * NC + lax.axis_index("c")
        pltpu.sync_copy(idx_hbm.at[pl.ds(wid * nch, nch)], idx_v)
        for j in range(nch):
            pltpu.async_copy(table_hbm.at[idx_v.at[j]], rows_v, sem).wait()
            pltpu.sync_copy(
                rows_v, out_hbm.at[pl.ds(wid * rpw + j * 128, 128)])

    return k(table, idx)


def kernel(queries, keys, values):
    Q, D = queries.shape
    NK = keys.shape[0]
    KP = ((NK + KB - 1) // KB) * KB
    keys_p = jnp.pad(keys, ((0, KP - NK), (0, 0)))
    w, gmax = _scores(queries, keys_p, NK)
    G = KP // GRP
    grow = _topgroups(gmax)
    cand4 = w.reshape(Q * (KP // 128), 128)[
        (grow >> 2).reshape(-1)].reshape(Q, TOPK, 128)
    topi, wn = _select(cand4, grow, G)
    gathered = _sc_gather(values, topi.reshape(Q * TOPK // 128, 128))
    out = jnp.einsum("qt,qtd->qd", wn, gathered.reshape(Q, TOPK, D))
    return out, topi
